# initial kernel scaffold (unmeasured)
import jax
import jax.numpy as jnp
from jax import lax
from jax.experimental import pallas as pl
from jax.experimental.pallas import tpu as pltpu

N_DEV = 32
R_HOPS = N_DEV // 2
L_HOPS = N_DEV // 2 - 1


def kernel(x, w_mat):
    m_per, k = x.shape
    _, n_per = w_mat.shape

    def body(x_ref, w_ref, out_ref,
             r_comm, l_comm, amax_send, amax_recv,
             r_send_sems, r_recv_sems, l_send_sems, l_recv_sems,
             amax_send_sems, amax_recv_sems):
        my_pos = lax.axis_index("i")
        left = lax.rem(my_pos - 1 + N_DEV, N_DEV)
        right = lax.rem(my_pos + 1, N_DEV)

        barrier_sem = pltpu.get_barrier_semaphore()
        for nbr in (left, right):
            pl.semaphore_signal(
                barrier_sem, inc=1,
                device_id=(nbr,), device_id_type=pl.DeviceIdType.MESH,
            )
        pl.semaphore_wait(barrier_sem, 2)

        def gemm(chunk):
            return lax.dot_general(
                chunk, w_ref[...],
                dimension_numbers=(((1,), (0,)), ((), ())),
                preferred_element_type=jnp.float32,
            )

        r_comm[0] = x_ref[...]
        l_comm[0] = x_ref[...]

        def make_rdma(comm, send_sems, recv_sems, h, dst):
            return pltpu.make_async_remote_copy(
                src_ref=comm.at[h % 2],
                dst_ref=comm.at[(h + 1) % 2],
                send_sem=send_sems.at[h % 2],
                recv_sem=recv_sems.at[(h + 1) % 2],
                device_id=(dst,),
                device_id_type=pl.DeviceIdType.MESH,
            )

        y0 = gemm(x_ref[...])
        out_ref[pl.ds(my_pos * m_per, m_per), :] = y0
        amax = jnp.maximum(jnp.max(y0), 0.0)

        for h in range(R_HOPS):
            r_rdma = make_rdma(r_comm, r_send_sems, r_recv_sems, h, right)
            r_rdma.start()
            if h < L_HOPS:
                l_rdma = make_rdma(l_comm, l_send_sems, l_recv_sems, h, left)
                l_rdma.start()

            r_rdma.wait()
            y = gemm(r_comm[(h + 1) % 2])
            origin = lax.rem(my_pos - (h + 1) + N_DEV, N_DEV)
            out_ref[pl.ds(origin * m_per, m_per), :] = y
            amax = jnp.maximum(amax, jnp.max(y))

            if h < L_HOPS:
                l_rdma.wait()
                y = gemm(l_comm[(h + 1) % 2])
                origin = lax.rem(my_pos + (h + 1), N_DEV)
                out_ref[pl.ds(origin * m_per, m_per), :] = y
                amax = jnp.maximum(amax, jnp.max(y))

        amax_send[...] = jnp.full(amax_send.shape, amax, jnp.float32)
        sends = []
        for o in range(1, N_DEV):
            dst = lax.rem(my_pos + o, N_DEV)
            rdma = pltpu.make_async_remote_copy(
                src_ref=amax_send,
                dst_ref=amax_recv.at[o],
                send_sem=amax_send_sems.at[o],
                recv_sem=amax_recv_sems.at[o],
                device_id=(dst,),
                device_id_type=pl.DeviceIdType.MESH,
            )
            rdma.start()
            sends.append(rdma)
        for rdma in sends:
            rdma.wait_send()
            rdma.wait_recv()

        g_amax = jnp.maximum(amax, jnp.max(amax_recv[1:, :, :]))

        scale = g_amax / 127.0
        y_all = jnp.maximum(out_ref[...], 0.0)
        q = jnp.clip(jnp.round(y_all / scale), 0.0, 127.0)
        out_ref[...] = q * scale

    grid_spec = pltpu.PrefetchScalarGridSpec(
        num_scalar_prefetch=0,
        grid=(),
        in_specs=[
            pl.BlockSpec(memory_space=pltpu.VMEM),
            pl.BlockSpec(memory_space=pltpu.VMEM),
        ],
        out_specs=pl.BlockSpec(memory_space=pltpu.VMEM),
        scratch_shapes=[
            pltpu.VMEM((2, m_per, k), jnp.float32),
            pltpu.VMEM((2, m_per, k), jnp.float32),
            pltpu.VMEM((8, 128), jnp.float32),
            pltpu.VMEM((N_DEV, 8, 128), jnp.float32),
            pltpu.SemaphoreType.DMA((2,)),
            pltpu.SemaphoreType.DMA((2,)),
            pltpu.SemaphoreType.DMA((2,)),
            pltpu.SemaphoreType.DMA((2,)),
            pltpu.SemaphoreType.DMA((N_DEV,)),
            pltpu.SemaphoreType.DMA((N_DEV,)),
        ],
    )
    return pl.pallas_call(
        body,
        out_shape=jax.ShapeDtypeStruct((N_DEV * m_per, n_per), jnp.float32),
        grid_spec=grid_spec,
        compiler_params=pltpu.CompilerParams(collective_id=0),
    )(x, w_mat)


# baseline (device time: 786777 ns/iter reference)
import jax
import jax.numpy as jnp
from jax import lax
from jax.experimental import pallas as pl
from jax.experimental.pallas import tpu as pltpu

N_DEV = 32
R_HOPS = N_DEV // 2
L_HOPS = N_DEV // 2 - 1
S = 4


def kernel(x, w_mat):
    m_per, k = x.shape
    _, n_per = w_mat.shape

    def body(x_ref, w_ref, out_ref,
             r_comm, l_comm, amax_send, amax_recv,
             r_send_sems, r_recv_sems, l_send_sems, l_recv_sems,
             amax_send_sems, amax_recv_sems, r_credit, l_credit):
        my_pos = lax.axis_index("i")
        left = lax.rem(my_pos - 1 + N_DEV, N_DEV)
        right = lax.rem(my_pos + 1, N_DEV)

        barrier_sem = pltpu.get_barrier_semaphore()
        for nbr in (left, right):
            pl.semaphore_signal(
                barrier_sem, inc=1,
                device_id=(nbr,), device_id_type=pl.DeviceIdType.MESH,
            )
        pl.semaphore_wait(barrier_sem, 2)

        def gemm(chunk):
            return lax.dot_general(
                chunk, w_ref[...],
                dimension_numbers=(((1,), (0,)), ((), ())),
                preferred_element_type=jnp.float32,
            )

        r_comm[0] = x_ref[...]
        l_comm[0] = x_ref[...]

        def make_rdma(comm, send_sems, recv_sems, h, dst):
            return pltpu.make_async_remote_copy(
                src_ref=comm.at[h % S],
                dst_ref=comm.at[(h + 1) % S],
                send_sem=send_sems.at[h % S],
                recv_sem=recv_sems.at[(h + 1) % S],
                device_id=(dst,),
                device_id_type=pl.DeviceIdType.MESH,
            )

        y0 = gemm(x_ref[...])
        out_ref[pl.ds(my_pos * m_per, m_per), :] = y0
        amax = jnp.maximum(jnp.max(y0), 0.0)

        for h in range(R_HOPS):
            if h >= S - 1:
                pl.semaphore_wait(r_credit, 1)
            r_rdma = make_rdma(r_comm, r_send_sems, r_recv_sems, h, right)
            r_rdma.start()
            if h < L_HOPS:
                if h >= S - 1:
                    pl.semaphore_wait(l_credit, 1)
                l_rdma = make_rdma(l_comm, l_send_sems, l_recv_sems, h, left)
                l_rdma.start()

            r_rdma.wait()
            if h <= R_HOPS - S:
                pl.semaphore_signal(
                    r_credit, inc=1,
                    device_id=(left,), device_id_type=pl.DeviceIdType.MESH,
                )
            y = gemm(r_comm[(h + 1) % S])
            origin = lax.rem(my_pos - (h + 1) + N_DEV, N_DEV)
            out_ref[pl.ds(origin * m_per, m_per), :] = y
            amax = jnp.maximum(amax, jnp.max(y))

            if h < L_HOPS:
                l_rdma.wait()
                if h <= L_HOPS - S:
                    pl.semaphore_signal(
                        l_credit, inc=1,
                        device_id=(right,), device_id_type=pl.DeviceIdType.MESH,
                    )
                y = gemm(l_comm[(h + 1) % S])
                origin = lax.rem(my_pos + (h + 1), N_DEV)
                out_ref[pl.ds(origin * m_per, m_per), :] = y
                amax = jnp.maximum(amax, jnp.max(y))

        amax_send[...] = jnp.full(amax_send.shape, amax, jnp.float32)
        sends = []
        for o in range(1, N_DEV):
            dst = lax.rem(my_pos + o, N_DEV)
            rdma = pltpu.make_async_remote_copy(
                src_ref=amax_send,
                dst_ref=amax_recv.at[o],
                send_sem=amax_send_sems.at[o],
                recv_sem=amax_recv_sems.at[o],
                device_id=(dst,),
                device_id_type=pl.DeviceIdType.MESH,
            )
            rdma.start()
            sends.append(rdma)
        for rdma in sends:
            rdma.wait_send()
            rdma.wait_recv()

        g_amax = jnp.maximum(amax, jnp.max(amax_recv[1:, :, :]))

        scale = g_amax / 127.0
        y_all = jnp.maximum(out_ref[...], 0.0)
        q = jnp.clip(jnp.round(y_all / scale), 0.0, 127.0)
        out_ref[...] = q * scale

    return pl.pallas_call(
        body,
        out_shape=jax.ShapeDtypeStruct((N_DEV * m_per, n_per), jnp.float32),
        in_specs=[
            pl.BlockSpec(memory_space=pltpu.VMEM),
            pl.BlockSpec(memory_space=pltpu.VMEM),
        ],
        out_specs=pl.BlockSpec(memory_space=pltpu.VMEM),
        scratch_shapes=[
            pltpu.VMEM((S, m_per, k), jnp.float32),
            pltpu.VMEM((S, m_per, k), jnp.float32),
            pltpu.VMEM((8, 128), jnp.float32),
            pltpu.VMEM((N_DEV, 8, 128), jnp.float32),
            pltpu.SemaphoreType.DMA((S,)),
            pltpu.SemaphoreType.DMA((S,)),
            pltpu.SemaphoreType.DMA((S,)),
            pltpu.SemaphoreType.DMA((S,)),
            pltpu.SemaphoreType.DMA((N_DEV,)),
            pltpu.SemaphoreType.DMA((N_DEV,)),
            pltpu.SemaphoreType.REGULAR,
            pltpu.SemaphoreType.REGULAR,
        ],
        compiler_params=pltpu.CompilerParams(collective_id=0),
    )(x, w_mat)


# device time: 760043 ns/iter; 1.0352x vs baseline; 1.0352x over previous
import jax
import jax.numpy as jnp
from jax import lax
from jax.experimental import pallas as pl
from jax.experimental.pallas import tpu as pltpu

N_DEV = 32
R_HOPS = N_DEV // 2
L_HOPS = N_DEV // 2 - 1
S = 4


def kernel(x, w_mat):
    m_per, k = x.shape
    _, n_per = w_mat.shape

    def body(x_ref, w_ref, out_ref,
             r_comm, l_comm, amax_send, amax_recv,
             r_send_sems, r_recv_sems, l_send_sems, l_recv_sems,
             amax_send_sems, amax_recv_sems, r_credit, l_credit):
        my_pos = lax.axis_index("i")
        left = lax.rem(my_pos - 1 + N_DEV, N_DEV)
        right = lax.rem(my_pos + 1, N_DEV)

        barrier_sem = pltpu.get_barrier_semaphore()
        for nbr in (left, right):
            pl.semaphore_signal(
                barrier_sem, inc=1,
                device_id=(nbr,), device_id_type=pl.DeviceIdType.MESH,
            )
        pl.semaphore_wait(barrier_sem, 2)

        def gemm(chunk):
            return lax.dot_general(
                chunk, w_ref[...],
                dimension_numbers=(((1,), (0,)), ((), ())),
                preferred_element_type=jnp.float32,
            )

        r_comm[0] = x_ref[...]
        l_comm[0] = x_ref[...]

        def make_rdma(comm, send_sems, recv_sems, h, dst):
            return pltpu.make_async_remote_copy(
                src_ref=comm.at[h % S],
                dst_ref=comm.at[(h + 1) % S],
                send_sem=send_sems.at[h % S],
                recv_sem=recv_sems.at[(h + 1) % S],
                device_id=(dst,),
                device_id_type=pl.DeviceIdType.MESH,
            )

        r_rd = [make_rdma(r_comm, r_send_sems, r_recv_sems, h, right)
                for h in range(R_HOPS)]
        l_rd = [make_rdma(l_comm, l_send_sems, l_recv_sems, h, left)
                for h in range(L_HOPS)]


        r_rd[0].start()
        l_rd[0].start()
        y0 = gemm(x_ref[...])
        out_ref[pl.ds(my_pos * m_per, m_per), :] = y0
        amax = jnp.maximum(jnp.max(y0), 0.0)
        r_rd[0].wait_send()
        pl.semaphore_signal(r_credit, inc=1, device_id=(left,),
                            device_id_type=pl.DeviceIdType.MESH)
        l_rd[0].wait_send()
        pl.semaphore_signal(l_credit, inc=1, device_id=(right,),
                            device_id_type=pl.DeviceIdType.MESH)

        for h in range(R_HOPS):
            if 1 <= h:
                r_rd[h].wait_send()
                if h <= R_HOPS - S:
                    pl.semaphore_signal(
                        r_credit, inc=1,
                        device_id=(left,), device_id_type=pl.DeviceIdType.MESH,
                    )
            if 1 <= h < L_HOPS:
                l_rd[h].wait_send()
                if h <= L_HOPS - S:
                    pl.semaphore_signal(
                        l_credit, inc=1,
                        device_id=(right,), device_id_type=pl.DeviceIdType.MESH,
                    )

            r_rd[h].wait_recv()
            if h + 1 < R_HOPS:
                if h + 1 >= S - 1:
                    pl.semaphore_wait(r_credit, 1)
                r_rd[h + 1].start()
            if h < L_HOPS:
                l_rd[h].wait_recv()
            if h + 1 < L_HOPS:
                if h + 1 >= S - 1:
                    pl.semaphore_wait(l_credit, 1)
                l_rd[h + 1].start()

            y = gemm(r_comm[(h + 1) % S])
            origin = lax.rem(my_pos - (h + 1) + N_DEV, N_DEV)
            out_ref[pl.ds(origin * m_per, m_per), :] = y
            amax = jnp.maximum(amax, jnp.max(y))

            if h < L_HOPS:
                y = gemm(l_comm[(h + 1) % S])
                origin = lax.rem(my_pos + (h + 1), N_DEV)
                out_ref[pl.ds(origin * m_per, m_per), :] = y
                amax = jnp.maximum(amax, jnp.max(y))

        amax_send[...] = jnp.full(amax_send.shape, amax, jnp.float32)
        sends = []
        for o in range(1, N_DEV):
            dst = lax.rem(my_pos + o, N_DEV)
            rdma = pltpu.make_async_remote_copy(
                src_ref=amax_send,
                dst_ref=amax_recv.at[o],
                send_sem=amax_send_sems.at[o],
                recv_sem=amax_recv_sems.at[o],
                device_id=(dst,),
                device_id_type=pl.DeviceIdType.MESH,
            )
            rdma.start()
            sends.append(rdma)
        for rdma in sends:
            rdma.wait_send()
            rdma.wait_recv()

        g_amax = jnp.maximum(amax, jnp.max(amax_recv[1:, :, :]))

        scale = g_amax / 127.0
        y_all = jnp.maximum(out_ref[...], 0.0)
        q = jnp.clip(jnp.round(y_all / scale), 0.0, 127.0)
        out_ref[...] = q * scale

    return pl.pallas_call(
        body,
        out_shape=jax.ShapeDtypeStruct((N_DEV * m_per, n_per), jnp.float32),
        in_specs=[
            pl.BlockSpec(memory_space=pltpu.VMEM),
            pl.BlockSpec(memory_space=pltpu.VMEM),
        ],
        out_specs=pl.BlockSpec(memory_space=pltpu.VMEM),
        scratch_shapes=[
            pltpu.VMEM((S, m_per, k), jnp.float32),
            pltpu.VMEM((S, m_per, k), jnp.float32),
            pltpu.VMEM((8, 128), jnp.float32),
            pltpu.VMEM((N_DEV, 8, 128), jnp.float32),
            pltpu.SemaphoreType.DMA((S,)),
            pltpu.SemaphoreType.DMA((S,)),
            pltpu.SemaphoreType.DMA((S,)),
            pltpu.SemaphoreType.DMA((S,)),
            pltpu.SemaphoreType.DMA((N_DEV,)),
            pltpu.SemaphoreType.DMA((N_DEV,)),
            pltpu.SemaphoreType.REGULAR,
            pltpu.SemaphoreType.REGULAR,
        ],
        compiler_params=pltpu.CompilerParams(collective_id=0),
    )(x, w_mat)


# device time: 405148 ns/iter; 1.9419x vs baseline; 1.8760x over previous
import jax
import jax.numpy as jnp
from jax import lax
from jax.experimental import pallas as pl
from jax.experimental.pallas import tpu as pltpu

N_DEV = 32
R_HOPS = N_DEV // 2
L_HOPS = N_DEV // 2 - 1
S = 4


def _build_cycle():
    order = [(0, 0), (1, 0), (1, 1), (0, 1), (0, 2), (1, 2), (1, 3), (0, 3)]

    def lid(x, y, z):
        return 8 * z + order.index((x, y))

    path = []
    for z in range(4):
        ys = range(4) if z % 2 == 0 else range(3, -1, -1)
        for y in ys:
            path.append((y, z))
    cyc = [lid(0, y, z) for (y, z) in path]
    cyc += [lid(1, y, z) for (y, z) in reversed(path)]
    return cyc


CYC = _build_cycle()
CYCIDX = [CYC.index(l) for l in range(N_DEV)]


def kernel(x, w_mat):
    m_per, k = x.shape
    _, n_per = w_mat.shape

    def body(x_ref, w_ref, out_ref,
             r_comm, l_comm, amax_send, amax_recv,
             r_send_sems, r_recv_sems, l_send_sems, l_recv_sems,
             amax_send_sems, amax_recv_sems, r_credit, l_credit):
        my_pos = lax.axis_index("i")

        def lut(table, idx):
            r = jnp.int32(table[0])
            for i in range(1, N_DEV):
                r = jnp.where(idx == i, jnp.int32(table[i]), r)
            return r

        ci = lut(CYCIDX, my_pos)
        left = lut(CYC, lax.rem(ci - 1 + N_DEV, N_DEV))
        right = lut(CYC, lax.rem(ci + 1, N_DEV))

        barrier_sem = pltpu.get_barrier_semaphore()
        for nbr in (left, right):
            pl.semaphore_signal(
                barrier_sem, inc=1,
                device_id=(nbr,), device_id_type=pl.DeviceIdType.MESH,
            )
        pl.semaphore_wait(barrier_sem, 2)

        def gemm(chunk):
            return lax.dot_general(
                chunk, w_ref[...],
                dimension_numbers=(((1,), (0,)), ((), ())),
                preferred_element_type=jnp.float32,
            )

        r_comm[0] = x_ref[...]
        l_comm[0] = x_ref[...]

        def make_rdma(comm, send_sems, recv_sems, h, dst):
            return pltpu.make_async_remote_copy(
                src_ref=comm.at[h % S],
                dst_ref=comm.at[(h + 1) % S],
                send_sem=send_sems.at[h % S],
                recv_sem=recv_sems.at[(h + 1) % S],
                device_id=(dst,),
                device_id_type=pl.DeviceIdType.MESH,
            )

        r_rd = [make_rdma(r_comm, r_send_sems, r_recv_sems, h, right)
                for h in range(R_HOPS)]
        l_rd = [make_rdma(l_comm, l_send_sems, l_recv_sems, h, left)
                for h in range(L_HOPS)]


        r_rd[0].start()
        l_rd[0].start()
        y0 = gemm(x_ref[...])
        out_ref[pl.ds(my_pos * m_per, m_per), :] = y0
        amax = jnp.maximum(jnp.max(y0), 0.0)
        r_rd[0].wait_send()
        pl.semaphore_signal(r_credit, inc=1, device_id=(left,),
                            device_id_type=pl.DeviceIdType.MESH)
        l_rd[0].wait_send()
        pl.semaphore_signal(l_credit, inc=1, device_id=(right,),
                            device_id_type=pl.DeviceIdType.MESH)

        for h in range(R_HOPS):
            if 1 <= h:
                r_rd[h].wait_send()
                if h <= R_HOPS - S:
                    pl.semaphore_signal(
                        r_credit, inc=1,
                        device_id=(left,), device_id_type=pl.DeviceIdType.MESH,
                    )
            if 1 <= h < L_HOPS:
                l_rd[h].wait_send()
                if h <= L_HOPS - S:
                    pl.semaphore_signal(
                        l_credit, inc=1,
                        device_id=(right,), device_id_type=pl.DeviceIdType.MESH,
                    )

            r_rd[h].wait_recv()
            if h + 1 < R_HOPS:
                if h + 1 >= S - 1:
                    pl.semaphore_wait(r_credit, 1)
                r_rd[h + 1].start()
            if h < L_HOPS:
                l_rd[h].wait_recv()
            if h + 1 < L_HOPS:
                if h + 1 >= S - 1:
                    pl.semaphore_wait(l_credit, 1)
                l_rd[h + 1].start()

            y = gemm(r_comm[(h + 1) % S])
            origin = lut(CYC, lax.rem(ci - (h + 1) + N_DEV, N_DEV))
            out_ref[pl.ds(origin * m_per, m_per), :] = y
            amax = jnp.maximum(amax, jnp.max(y))

            if h < L_HOPS:
                y = gemm(l_comm[(h + 1) % S])
                origin = lut(CYC, lax.rem(ci + (h + 1), N_DEV))
                out_ref[pl.ds(origin * m_per, m_per), :] = y
                amax = jnp.maximum(amax, jnp.max(y))

        amax_send[...] = jnp.full(amax_send.shape, amax, jnp.float32)
        sends = []
        for o in range(1, N_DEV):
            dst = lax.rem(my_pos + o, N_DEV)
            rdma = pltpu.make_async_remote_copy(
                src_ref=amax_send,
                dst_ref=amax_recv.at[o],
                send_sem=amax_send_sems.at[o],
                recv_sem=amax_recv_sems.at[o],
                device_id=(dst,),
                device_id_type=pl.DeviceIdType.MESH,
            )
            rdma.start()
            sends.append(rdma)
        for rdma in sends:
            rdma.wait_send()
            rdma.wait_recv()

        g_amax = jnp.maximum(amax, jnp.max(amax_recv[1:, :, :]))

        scale = g_amax / 127.0
        y_all = jnp.maximum(out_ref[...], 0.0)
        q = jnp.clip(jnp.round(y_all / scale), 0.0, 127.0)
        out_ref[...] = q * scale

    return pl.pallas_call(
        body,
        out_shape=jax.ShapeDtypeStruct((N_DEV * m_per, n_per), jnp.float32),
        in_specs=[
            pl.BlockSpec(memory_space=pltpu.VMEM),
            pl.BlockSpec(memory_space=pltpu.VMEM),
        ],
        out_specs=pl.BlockSpec(memory_space=pltpu.VMEM),
        scratch_shapes=[
            pltpu.VMEM((S, m_per, k), jnp.float32),
            pltpu.VMEM((S, m_per, k), jnp.float32),
            pltpu.VMEM((8, 128), jnp.float32),
            pltpu.VMEM((N_DEV, 8, 128), jnp.float32),
            pltpu.SemaphoreType.DMA((S,)),
            pltpu.SemaphoreType.DMA((S,)),
            pltpu.SemaphoreType.DMA((S,)),
            pltpu.SemaphoreType.DMA((S,)),
            pltpu.SemaphoreType.DMA((N_DEV,)),
            pltpu.SemaphoreType.DMA((N_DEV,)),
            pltpu.SemaphoreType.REGULAR,
            pltpu.SemaphoreType.REGULAR,
        ],
        compiler_params=pltpu.CompilerParams(collective_id=0),
    )(x, w_mat)


# device time: 386475 ns/iter; 2.0358x vs baseline; 1.0483x over previous
import jax
import jax.numpy as jnp
from jax import lax
from jax.experimental import pallas as pl
from jax.experimental.pallas import tpu as pltpu

N_DEV = 32
HOPS = N_DEV // 2
S = 4
HALF = None


def _build_cycle():
    order = [(0, 0), (1, 0), (1, 1), (0, 1), (0, 2), (1, 2), (1, 3), (0, 3)]

    def lid(x, y, z):
        return 8 * z + order.index((x, y))

    path = []
    for z in range(4):
        ys = range(4) if z % 2 == 0 else range(3, -1, -1)
        for y in ys:
            path.append((y, z))
    cyc = [lid(0, y, z) for (y, z) in path]
    cyc += [lid(1, y, z) for (y, z) in reversed(path)]
    return cyc


CYC = _build_cycle()
CYCIDX = [CYC.index(l) for l in range(N_DEV)]


def kernel(x, w_mat):
    m_per, k = x.shape
    _, n_per = w_mat.shape
    half = m_per // 2

    RA_H, RB_H = HOPS, HOPS - 1
    LB_H, LA_H = HOPS, HOPS - 1

    def body(x_ref, w_ref, out_ref,
             r_comm, l_comm, amax_send, amax_recv,
             ra_s, ra_r, rb_s, rb_r, la_s, la_r, lb_s, lb_r,
             amax_send_sems, amax_recv_sems, r_credit, l_credit):
        my_pos = lax.axis_index("i")

        def lut(table, idx):
            r = jnp.int32(table[0])
            for i in range(1, N_DEV):
                r = jnp.where(idx == i, jnp.int32(table[i]), r)
            return r

        ci = lut(CYCIDX, my_pos)
        left = lut(CYC, lax.rem(ci - 1 + N_DEV, N_DEV))
        right = lut(CYC, lax.rem(ci + 1, N_DEV))

        barrier_sem = pltpu.get_barrier_semaphore()
        for nbr in (left, right):
            pl.semaphore_signal(
                barrier_sem, inc=1,
                device_id=(nbr,), device_id_type=pl.DeviceIdType.MESH,
            )
        pl.semaphore_wait(barrier_sem, 2)

        def gemm(chunk):
            return lax.dot_general(
                chunk, w_ref[...],
                dimension_numbers=(((1,), (0,)), ((), ())),
                preferred_element_type=jnp.float32,
            )

        def make_half(comm, send_sems, recv_sems, h, dst, row0):
            return pltpu.make_async_remote_copy(
                src_ref=comm.at[h % S, pl.ds(row0, half)],
                dst_ref=comm.at[(h + 1) % S, pl.ds(row0, half)],
                send_sem=send_sems.at[h % S],
                recv_sem=recv_sems.at[(h + 1) % S],
                device_id=(dst,),
                device_id_type=pl.DeviceIdType.MESH,
            )

        rA = [make_half(r_comm, ra_s, ra_r, h, right, 0) for h in range(RA_H)]
        rB = [make_half(r_comm, rb_s, rb_r, h, right, half) for h in range(RB_H)]
        lB = [make_half(l_comm, lb_s, lb_r, h, left, half) for h in range(LB_H)]
        lA = [make_half(l_comm, la_s, la_r, h, left, 0) for h in range(LA_H)]

        def credit_signal(sem, dst):
            pl.semaphore_signal(sem, inc=1, device_id=(dst,),
                                device_id_type=pl.DeviceIdType.MESH)

        r_comm[0] = x_ref[...]
        l_comm[0] = x_ref[...]
        rA[0].start()
        rB[0].start()
        lB[0].start()
        lA[0].start()
        y0 = gemm(x_ref[...])
        out_ref[pl.ds(my_pos * m_per, m_per), :] = y0
        amax = jnp.maximum(jnp.max(y0), 0.0)
        rA[0].wait_send()
        rB[0].wait_send()
        credit_signal(r_credit, left)
        lB[0].wait_send()
        lA[0].wait_send()
        credit_signal(l_credit, right)

        for h in range(HOPS):
            if h >= 1:
                rA[h].wait_send()
                if h < RB_H:
                    rB[h].wait_send()
                if h <= RA_H - S:
                    credit_signal(r_credit, left)
                lB[h].wait_send()
                if h < LA_H:
                    lA[h].wait_send()
                if h <= LB_H - S:
                    credit_signal(l_credit, right)

            rA[h].wait_recv()
            if h + 1 < RA_H:
                if h + 1 >= S - 1:
                    pl.semaphore_wait(r_credit, 1)
                rA[h + 1].start()
            if h < RB_H:
                rB[h].wait_recv()
                if h + 1 < RB_H:
                    rB[h + 1].start()
            lB[h].wait_recv()
            if h + 1 < LB_H:
                if h + 1 >= S - 1:
                    pl.semaphore_wait(l_credit, 1)
                lB[h + 1].start()
            if h < LA_H:
                lA[h].wait_recv()
                if h + 1 < LA_H:
                    lA[h + 1].start()

            slot = (h + 1) % S
            if h < HOPS - 1:
                y = gemm(r_comm[slot])
                origin = lut(CYC, lax.rem(ci - (h + 1) + N_DEV, N_DEV))
                out_ref[pl.ds(origin * m_per, m_per), :] = y
                amax = jnp.maximum(amax, jnp.max(y))
                y = gemm(l_comm[slot])
                origin = lut(CYC, lax.rem(ci + (h + 1), N_DEV))
                out_ref[pl.ds(origin * m_per, m_per), :] = y
                amax = jnp.maximum(amax, jnp.max(y))
            else:
                origin = lut(CYC, lax.rem(ci + HOPS, N_DEV))
                y = gemm(r_comm[slot, 0:half])
                out_ref[pl.ds(origin * m_per, half), :] = y
                amax = jnp.maximum(amax, jnp.max(y))
                y = gemm(l_comm[slot, half:m_per])
                out_ref[pl.ds(origin * m_per + half, half), :] = y
                amax = jnp.maximum(amax, jnp.max(y))

        amax_send[...] = jnp.full(amax_send.shape, amax, jnp.float32)
        sends = []
        for o in range(1, N_DEV):
            dst = lax.rem(my_pos + o, N_DEV)
            rdma = pltpu.make_async_remote_copy(
                src_ref=amax_send,
                dst_ref=amax_recv.at[o],
                send_sem=amax_send_sems.at[o],
                recv_sem=amax_recv_sems.at[o],
                device_id=(dst,),
                device_id_type=pl.DeviceIdType.MESH,
            )
            rdma.start()
            sends.append(rdma)
        for rdma in sends:
            rdma.wait_send()
            rdma.wait_recv()

        g_amax = jnp.maximum(amax, jnp.max(amax_recv[1:, :, :]))

        scale = g_amax / 127.0
        y_all = jnp.maximum(out_ref[...], 0.0)
        q = jnp.clip(jnp.round(y_all / scale), 0.0, 127.0)
        out_ref[...] = q * scale

    return pl.pallas_call(
        body,
        out_shape=jax.ShapeDtypeStruct((N_DEV * m_per, n_per), jnp.float32),
        in_specs=[
            pl.BlockSpec(memory_space=pltpu.VMEM),
            pl.BlockSpec(memory_space=pltpu.VMEM),
        ],
        out_specs=pl.BlockSpec(memory_space=pltpu.VMEM),
        scratch_shapes=[
            pltpu.VMEM((S, m_per, k), jnp.float32),
            pltpu.VMEM((S, m_per, k), jnp.float32),
            pltpu.VMEM((8, 128), jnp.float32),
            pltpu.VMEM((N_DEV, 8, 128), jnp.float32),
            pltpu.SemaphoreType.DMA((S,)),
            pltpu.SemaphoreType.DMA((S,)),
            pltpu.SemaphoreType.DMA((S,)),
            pltpu.SemaphoreType.DMA((S,)),
            pltpu.SemaphoreType.DMA((S,)),
            pltpu.SemaphoreType.DMA((S,)),
            pltpu.SemaphoreType.DMA((S,)),
            pltpu.SemaphoreType.DMA((S,)),
            pltpu.SemaphoreType.DMA((N_DEV,)),
            pltpu.SemaphoreType.DMA((N_DEV,)),
            pltpu.SemaphoreType.REGULAR,
            pltpu.SemaphoreType.REGULAR,
        ],
        compiler_params=pltpu.CompilerParams(collective_id=0),
    )(x, w_mat)


# device time: 212567 ns/iter; 3.7013x vs baseline; 1.8181x over previous
import jax
import jax.numpy as jnp
from jax import lax
from jax.experimental import pallas as pl
from jax.experimental.pallas import tpu as pltpu

N_DEV = 32
HOPS = N_DEV // 2
S = 4
HALF = None


def _build_cycle():
    order = [(0, 0), (1, 0), (1, 1), (0, 1), (0, 2), (1, 2), (1, 3), (0, 3)]

    def lid(x, y, z):
        return 8 * z + order.index((x, y))

    path = []
    for z in range(4):
        ys = range(4) if z % 2 == 0 else range(3, -1, -1)
        for y in ys:
            path.append((y, z))
    cyc = [lid(0, y, z) for (y, z) in path]
    cyc += [lid(1, y, z) for (y, z) in reversed(path)]
    return cyc


CYC = _build_cycle()
CYCIDX = [CYC.index(l) for l in range(N_DEV)]


def kernel(x, w_mat):
    m_per, k = x.shape
    _, n_per = w_mat.shape
    half = m_per // 2

    RA_H, RB_H = HOPS, HOPS - 1
    LB_H, LA_H = HOPS, HOPS - 1

    def body(x_ref, w_ref, out_ref,
             r_comm, l_comm, w_bf, amax_send, amax_recv,
             ra_s, ra_r, rb_s, rb_r, la_s, la_r, lb_s, lb_r,
             amax_send_sems, amax_recv_sems, r_credit, l_credit):
        my_pos = lax.axis_index("i")

        def lut(table, idx):
            r = jnp.int32(table[0])
            for i in range(1, N_DEV):
                r = jnp.where(idx == i, jnp.int32(table[i]), r)
            return r

        ci = lut(CYCIDX, my_pos)
        left = lut(CYC, lax.rem(ci - 1 + N_DEV, N_DEV))
        right = lut(CYC, lax.rem(ci + 1, N_DEV))

        barrier_sem = pltpu.get_barrier_semaphore()
        for nbr in (left, right):
            pl.semaphore_signal(
                barrier_sem, inc=1,
                device_id=(nbr,), device_id_type=pl.DeviceIdType.MESH,
            )
        pl.semaphore_wait(barrier_sem, 2)

        w_bf[...] = w_ref[...].astype(jnp.bfloat16)

        def gemm(chunk):
            return lax.dot_general(
                chunk, w_bf[...],
                dimension_numbers=(((1,), (0,)), ((), ())),
                preferred_element_type=jnp.float32,
            )

        def make_half(comm, send_sems, recv_sems, h, dst, row0):
            return pltpu.make_async_remote_copy(
                src_ref=comm.at[h % S, pl.ds(row0, half)],
                dst_ref=comm.at[(h + 1) % S, pl.ds(row0, half)],
                send_sem=send_sems.at[h % S],
                recv_sem=recv_sems.at[(h + 1) % S],
                device_id=(dst,),
                device_id_type=pl.DeviceIdType.MESH,
            )

        rA = [make_half(r_comm, ra_s, ra_r, h, right, 0) for h in range(RA_H)]
        rB = [make_half(r_comm, rb_s, rb_r, h, right, half) for h in range(RB_H)]
        lB = [make_half(l_comm, lb_s, lb_r, h, left, half) for h in range(LB_H)]
        lA = [make_half(l_comm, la_s, la_r, h, left, 0) for h in range(LA_H)]

        def credit_signal(sem, dst):
            pl.semaphore_signal(sem, inc=1, device_id=(dst,),
                                device_id_type=pl.DeviceIdType.MESH)

        x_bf = x_ref[...].astype(jnp.bfloat16)
        r_comm[0] = x_bf
        l_comm[0] = x_bf
        rA[0].start()
        rB[0].start()
        lB[0].start()
        lA[0].start()
        y0 = gemm(x_bf)
        out_ref[pl.ds(my_pos * m_per, m_per), :] = y0
        amax = jnp.maximum(jnp.max(y0), 0.0)
        rA[0].wait_send()
        rB[0].wait_send()
        credit_signal(r_credit, left)
        lB[0].wait_send()
        lA[0].wait_send()
        credit_signal(l_credit, right)

        for h in range(HOPS):
            if h >= 1:
                rA[h].wait_send()
                if h < RB_H:
                    rB[h].wait_send()
                if h <= RA_H - S:
                    credit_signal(r_credit, left)
                lB[h].wait_send()
                if h < LA_H:
                    lA[h].wait_send()
                if h <= LB_H - S:
                    credit_signal(l_credit, right)

            rA[h].wait_recv()
            if h + 1 < RA_H:
                if h + 1 >= S - 1:
                    pl.semaphore_wait(r_credit, 1)
                rA[h + 1].start()
            if h < RB_H:
                rB[h].wait_recv()
                if h + 1 < RB_H:
                    rB[h + 1].start()
            lB[h].wait_recv()
            if h + 1 < LB_H:
                if h + 1 >= S - 1:
                    pl.semaphore_wait(l_credit, 1)
                lB[h + 1].start()
            if h < LA_H:
                lA[h].wait_recv()
                if h + 1 < LA_H:
                    lA[h + 1].start()

            slot = (h + 1) % S
            if h < HOPS - 1:
                y = gemm(r_comm[slot])
                origin = lut(CYC, lax.rem(ci - (h + 1) + N_DEV, N_DEV))
                out_ref[pl.ds(origin * m_per, m_per), :] = y
                amax = jnp.maximum(amax, jnp.max(y))
                y = gemm(l_comm[slot])
                origin = lut(CYC, lax.rem(ci + (h + 1), N_DEV))
                out_ref[pl.ds(origin * m_per, m_per), :] = y
                amax = jnp.maximum(amax, jnp.max(y))
            else:
                origin = lut(CYC, lax.rem(ci + HOPS, N_DEV))
                y = gemm(r_comm[slot, 0:half])
                out_ref[pl.ds(origin * m_per, half), :] = y
                amax = jnp.maximum(amax, jnp.max(y))
                y = gemm(l_comm[slot, half:m_per])
                out_ref[pl.ds(origin * m_per + half, half), :] = y
                amax = jnp.maximum(amax, jnp.max(y))

        amax_send[...] = jnp.full(amax_send.shape, amax, jnp.float32)
        sends = []
        for o in range(1, N_DEV):
            dst = lax.rem(my_pos + o, N_DEV)
            rdma = pltpu.make_async_remote_copy(
                src_ref=amax_send,
                dst_ref=amax_recv.at[o],
                send_sem=amax_send_sems.at[o],
                recv_sem=amax_recv_sems.at[o],
                device_id=(dst,),
                device_id_type=pl.DeviceIdType.MESH,
            )
            rdma.start()
            sends.append(rdma)
        for rdma in sends:
            rdma.wait_send()
            rdma.wait_recv()

        g_amax = jnp.maximum(amax, jnp.max(amax_recv[1:, :, :]))

        scale = g_amax / 127.0
        inv = 127.0 / g_amax
        y_all = jnp.maximum(out_ref[...], 0.0)
        q = jnp.clip(jnp.round(y_all * inv), 0.0, 127.0)
        out_ref[...] = q * scale

    return pl.pallas_call(
        body,
        out_shape=jax.ShapeDtypeStruct((N_DEV * m_per, n_per), jnp.float32),
        in_specs=[
            pl.BlockSpec(memory_space=pltpu.VMEM),
            pl.BlockSpec(memory_space=pltpu.VMEM),
        ],
        out_specs=pl.BlockSpec(memory_space=pltpu.VMEM),
        scratch_shapes=[
            pltpu.VMEM((S, m_per, k), jnp.bfloat16),
            pltpu.VMEM((S, m_per, k), jnp.bfloat16),
            pltpu.VMEM((k, n_per), jnp.bfloat16),
            pltpu.VMEM((8, 128), jnp.float32),
            pltpu.VMEM((N_DEV, 8, 128), jnp.float32),
            pltpu.SemaphoreType.DMA((S,)),
            pltpu.SemaphoreType.DMA((S,)),
            pltpu.SemaphoreType.DMA((S,)),
            pltpu.SemaphoreType.DMA((S,)),
            pltpu.SemaphoreType.DMA((S,)),
            pltpu.SemaphoreType.DMA((S,)),
            pltpu.SemaphoreType.DMA((S,)),
            pltpu.SemaphoreType.DMA((S,)),
            pltpu.SemaphoreType.DMA((N_DEV,)),
            pltpu.SemaphoreType.DMA((N_DEV,)),
            pltpu.SemaphoreType.REGULAR,
            pltpu.SemaphoreType.REGULAR,
        ],
        compiler_params=pltpu.CompilerParams(collective_id=0),
    )(x, w_mat)


# device time: 212168 ns/iter; 3.7083x vs baseline; 1.0019x over previous
import jax
import jax.numpy as jnp
from jax import lax
from jax.experimental import pallas as pl
from jax.experimental.pallas import tpu as pltpu

N_DEV = 32
HOPS = N_DEV // 2
S = 4


def _build_cycle():
    order = [(0, 0), (1, 0), (1, 1), (0, 1), (0, 2), (1, 2), (1, 3), (0, 3)]

    def lid(x, y, z):
        return 8 * z + order.index((x, y))

    path = []
    for z in range(4):
        ys = range(4) if z % 2 == 0 else range(3, -1, -1)
        for y in ys:
            path.append((y, z))
    cyc = [lid(0, y, z) for (y, z) in path]
    cyc += [lid(1, y, z) for (y, z) in reversed(path)]
    return cyc


CYC = _build_cycle()
CYCIDX = [CYC.index(l) for l in range(N_DEV)]


def kernel(x, w_mat):
    m_per, k = x.shape
    _, n_per = w_mat.shape
    half = m_per // 2

    RA_H, RB_H = HOPS, HOPS - 1
    LB_H, LA_H = HOPS, HOPS - 1

    def body(x_ref, w_ref, out_ref,
             r_comm, l_comm, w_bf, amax_send, amax_recv,
             ra_s, ra_r, rb_s, rb_r, la_s, la_r, lb_s, lb_r,
             amax_send_sems, amax_recv_sems, r_credit, l_credit):
        my_pos = lax.axis_index("i")

        def lut(table, idx):
            r = jnp.int32(table[0])
            for i in range(1, N_DEV):
                r = jnp.where(idx == i, jnp.int32(table[i]), r)
            return r

        ci = lut(CYCIDX, my_pos)
        left = lut(CYC, lax.rem(ci - 1 + N_DEV, N_DEV))
        right = lut(CYC, lax.rem(ci + 1, N_DEV))

        barrier_sem = pltpu.get_barrier_semaphore()
        for nbr in (left, right):
            pl.semaphore_signal(
                barrier_sem, inc=1,
                device_id=(nbr,), device_id_type=pl.DeviceIdType.MESH,
            )
        pl.semaphore_wait(barrier_sem, 2)

        w_bf[...] = w_ref[...].astype(jnp.bfloat16)

        def gemm(chunk):
            return lax.dot_general(
                chunk, w_bf[...],
                dimension_numbers=(((1,), (0,)), ((), ())),
                preferred_element_type=jnp.float32,
            )

        def make_half(comm, send_sems, recv_sems, h, dst, row0):
            return pltpu.make_async_remote_copy(
                src_ref=comm.at[h % S, pl.ds(row0, half)],
                dst_ref=comm.at[(h + 1) % S, pl.ds(row0, half)],
                send_sem=send_sems.at[h % S],
                recv_sem=recv_sems.at[(h + 1) % S],
                device_id=(dst,),
                device_id_type=pl.DeviceIdType.MESH,
            )

        rA = [make_half(r_comm, ra_s, ra_r, h, right, 0) for h in range(RA_H)]
        rB = [make_half(r_comm, rb_s, rb_r, h, right, half) for h in range(RB_H)]
        lB = [make_half(l_comm, lb_s, lb_r, h, left, half) for h in range(LB_H)]
        lA = [make_half(l_comm, la_s, la_r, h, left, 0) for h in range(LA_H)]

        def credit_signal(sem, dst):
            pl.semaphore_signal(sem, inc=1, device_id=(dst,),
                                device_id_type=pl.DeviceIdType.MESH)

        x_bf = x_ref[...].astype(jnp.bfloat16)
        r_comm[0] = x_bf
        l_comm[0] = x_bf
        rA[0].start()
        rB[0].start()
        lB[0].start()
        lA[0].start()
        y0 = gemm(x_bf)
        out_ref[pl.ds(my_pos * m_per, m_per), :] = y0
        amax = jnp.maximum(jnp.max(y0), 0.0)
        rA[0].wait_send()
        rB[0].wait_send()
        credit_signal(r_credit, left)
        lB[0].wait_send()
        lA[0].wait_send()
        credit_signal(l_credit, right)

        for h in range(HOPS):
            if h >= 1:
                rA[h].wait_send()
                if h < RB_H:
                    rB[h].wait_send()
                if h <= RA_H - S:
                    credit_signal(r_credit, left)
                lB[h].wait_send()
                if h < LA_H:
                    lA[h].wait_send()
                if h <= LB_H - S:
                    credit_signal(l_credit, right)

            rA[h].wait_recv()
            if h + 1 < RA_H:
                if h + 1 >= S - 1:
                    pl.semaphore_wait(r_credit, 1)
                rA[h + 1].start()
            if h < RB_H:
                rB[h].wait_recv()
                if h + 1 < RB_H:
                    rB[h + 1].start()
            lB[h].wait_recv()
            if h + 1 < LB_H:
                if h + 1 >= S - 1:
                    pl.semaphore_wait(l_credit, 1)
                lB[h + 1].start()
            if h < LA_H:
                lA[h].wait_recv()
                if h + 1 < LA_H:
                    lA[h + 1].start()

            slot = (h + 1) % S
            if h < HOPS - 1:
                y = gemm(r_comm[slot])
                origin = lut(CYC, lax.rem(ci - (h + 1) + N_DEV, N_DEV))
                out_ref[pl.ds(origin * m_per, m_per), :] = y
                amax = jnp.maximum(amax, jnp.max(y))
                y = gemm(l_comm[slot])
                origin = lut(CYC, lax.rem(ci + (h + 1), N_DEV))
                out_ref[pl.ds(origin * m_per, m_per), :] = y
                amax = jnp.maximum(amax, jnp.max(y))
            else:
                origin = lut(CYC, lax.rem(ci + HOPS, N_DEV))
                y = gemm(r_comm[slot, 0:half])
                out_ref[pl.ds(origin * m_per, half), :] = y
                amax = jnp.maximum(amax, jnp.max(y))
                y = gemm(l_comm[slot, half:m_per])
                out_ref[pl.ds(origin * m_per + half, half), :] = y
                amax = jnp.maximum(amax, jnp.max(y))

        amax_send[...] = jnp.full(amax_send.shape, amax, jnp.float32)
        sends = []
        for o in range(1, N_DEV):
            dst = lax.rem(my_pos + o, N_DEV)
            rdma = pltpu.make_async_remote_copy(
                src_ref=amax_send,
                dst_ref=amax_recv.at[o],
                send_sem=amax_send_sems.at[o],
                recv_sem=amax_recv_sems.at[o],
                device_id=(dst,),
                device_id_type=pl.DeviceIdType.MESH,
            )
            rdma.start()
            sends.append(rdma)
        for rdma in sends:
            rdma.wait_send()
            rdma.wait_recv()

        g_amax = jnp.maximum(amax, jnp.max(amax_recv[1:, :, :]))

        scale = g_amax / 127.0
        inv = 127.0 / g_amax
        q = jnp.clip(jnp.round(out_ref[...] * inv), 0.0, 127.0)
        out_ref[...] = q * scale

    return pl.pallas_call(
        body,
        out_shape=jax.ShapeDtypeStruct((N_DEV * m_per, n_per), jnp.float32),
        in_specs=[
            pl.BlockSpec(memory_space=pltpu.VMEM),
            pl.BlockSpec(memory_space=pltpu.VMEM),
        ],
        out_specs=pl.BlockSpec(memory_space=pltpu.VMEM),
        scratch_shapes=[
            pltpu.VMEM((S, m_per, k), jnp.bfloat16),
            pltpu.VMEM((S, m_per, k), jnp.bfloat16),
            pltpu.VMEM((k, n_per), jnp.bfloat16),
            pltpu.VMEM((8, 128), jnp.float32),
            pltpu.VMEM((N_DEV, 8, 128), jnp.float32),
            pltpu.SemaphoreType.DMA((S,)),
            pltpu.SemaphoreType.DMA((S,)),
            pltpu.SemaphoreType.DMA((S,)),
            pltpu.SemaphoreType.DMA((S,)),
            pltpu.SemaphoreType.DMA((S,)),
            pltpu.SemaphoreType.DMA((S,)),
            pltpu.SemaphoreType.DMA((S,)),
            pltpu.SemaphoreType.DMA((S,)),
            pltpu.SemaphoreType.DMA((N_DEV,)),
            pltpu.SemaphoreType.DMA((N_DEV,)),
            pltpu.SemaphoreType.REGULAR,
            pltpu.SemaphoreType.REGULAR,
        ],
        compiler_params=pltpu.CompilerParams(collective_id=0),
    )(x, w_mat)
